# baseline (speedup 1.0000x reference)
import jax
import jax.numpy as jnp
from jax import lax
from jax.experimental import pallas as pl
from jax.experimental.pallas import tpu as pltpu
from jax.experimental.pallas import tpu_sc as plsc

NC, NS = 2, 16
NW = NC * NS
B = 16384
BPW = B // NW
WD, RD = 64, 16
OD = WD + RD + 1
L = 16
G = BPW // L
NRING = 8


def _body(weapon_hbm, rank_hbm, level_hbm, wtab_t_hbm, rtab2_hbm, out_hbm,
          idx_w, idx_r, lvl, rtab_v, ring, rowbuf, sems):
    wid = lax.axis_index("s") * NC + lax.axis_index("c")
    base = wid * BPW
    rows_out = pl.ds(base, BPW)
    pltpu.sync_copy(weapon_hbm.at[rows_out], idx_w)
    pltpu.sync_copy(rank_hbm.at[rows_out], idx_r)
    pltpu.sync_copy(level_hbm.at[rows_out], lvl)
    pltpu.sync_copy(rtab2_hbm, rtab_v)

    lanes = lax.iota(jnp.int32, L)
    rows4 = [lanes + L * k for k in range(WD // L)]
    col80 = jnp.full((L,), WD + RD, dtype=jnp.int32)

    def start_fetch(ivec, l, slot):
        i = ivec[l]
        off = pl.multiple_of(lax.shift_right_logical(i, 7) * 128, 128)
        for gr in range(8):
            pltpu.async_copy(
                wtab_t_hbm.at[pl.ds(gr * 8, 8), pl.ds(off, 128)],
                ring.at[slot, pl.ds(gr * 8, 8)], sems[slot])

    ivec0 = idx_w[pl.ds(0, L)]
    for l in range(NRING):
        start_fetch(ivec0, l, l)

    def group_body(g, _):
        sl = pl.ds(g * L, L)
        ivec = idx_w[sl]
        gn = jnp.minimum(g + 1, G - 1)
        ivec_n = idx_w[pl.ds(gn * L, L)]
        for l in range(L):
            slot = l % NRING
            pltpu.make_async_copy(wtab_t_hbm.at[:, pl.ds(0, 128)],
                                  ring.at[slot], sems[slot]).wait()
            ccv = jnp.full((L,), lax.bitwise_and(ivec[l], 127), jnp.int32)
            for k in range(WD // L):
                rowbuf[l, pl.ds(k * L, L)] = plsc.load_gather(
                    ring.at[slot], [rows4[k], ccv])
            if l < L - NRING:
                start_fetch(ivec, l + NRING, slot)
            else:
                start_fetch(ivec_n, l + NRING - L, slot)
        ri = idx_r[sl]
        rrow = lax.shift_right_logical(ri, 3)
        rcol = lax.mul(lax.bitwise_and(ri, 7), RD)
        for c in range(RD):
            plsc.store_scatter(
                rowbuf, [lanes, jnp.full((L,), WD + c, jnp.int32)],
                plsc.load_gather(rtab_v, [rrow, rcol + c]))
        plsc.store_scatter(rowbuf, [lanes, col80], lvl[sl])
        pltpu.sync_copy(
            rowbuf, out_hbm.at[pl.ds(pl.multiple_of(base + g * L, L), L)])
        return 0

    lax.fori_loop(0, G, group_body, 0)
    for l in range(NRING):
        pltpu.make_async_copy(wtab_t_hbm.at[:, pl.ds(0, 128)],
                              ring.at[l], sems[l]).wait()


def kernel(weapon, rank, level, weapon_table, rank_table):
    wtab_t = weapon_table.T
    rtab2 = rank_table.reshape(125, 128)
    mesh = plsc.VectorSubcoreMesh(core_axis_name="c", subcore_axis_name="s")
    k = pl.kernel(
        _body,
        out_type=jax.ShapeDtypeStruct((B, 128), jnp.float32),
        mesh=mesh,
        scratch_types=[
            pltpu.VMEM((BPW,), jnp.int32),
            pltpu.VMEM((BPW,), jnp.int32),
            pltpu.VMEM((BPW,), jnp.float32),
            pltpu.VMEM((125, 128), jnp.float32),
            pltpu.VMEM((NRING, WD, 128), jnp.float32),
            pltpu.VMEM((L, 128), jnp.float32),
            [pltpu.SemaphoreType.DMA] * NRING,
        ],
        compiler_params=pltpu.CompilerParams(
            use_tc_tiling_on_sc=True, needs_layout_passes=False),
    )
    out128 = k(weapon, rank, level, wtab_t, rtab2)
    return out128[:, :OD]
